# Initial kernel scaffold; baseline (speedup 1.0000x reference)
import jax
import jax.numpy as jnp
from jax.experimental import pallas as pl


def kernel(token_ids, weight):
    raise NotImplementedError("write your pallas kernel here")


# speedup vs baseline: 2.0269x; 2.0269x over previous
import functools

import jax
import jax.numpy as jnp
from jax import lax
from jax.experimental import pallas as pl
from jax.experimental.pallas import tpu as pltpu
from jax.experimental.pallas import tpu_sc as plsc

_NC = 2
_NS = 16
_NW = _NC * _NS

_KK = 5
_NB = 2


def _embed_sc(tok2, weight, R, D):
    rows_per_w = R // _NW
    steps = rows_per_w // _KK
    niter = steps // _NB

    mesh = plsc.VectorSubcoreMesh(core_axis_name="c", subcore_axis_name="s")

    @functools.partial(
        pl.kernel,
        mesh=mesh,
        compiler_params=pltpu.CompilerParams(use_tc_tiling_on_sc=False),
        out_type=jax.ShapeDtypeStruct((R, 128, D), jnp.float32),
        scratch_types=[
            pltpu.VMEM((rows_per_w, 128), jnp.int32),
            pltpu.VMEM((_NB, _KK, 128, D), jnp.float32),
            pltpu.SemaphoreType.DMA((_NB,)),
            pltpu.SemaphoreType.DMA((_NB,)),
        ],
    )
    def emb(tok_hbm, w_hbm, out_hbm, idx_all, row_b, gsem, osem):
        wid = lax.axis_index("s") * _NC + lax.axis_index("c")
        base = wid * rows_per_w

        pltpu.sync_copy(tok_hbm.at[pl.ds(base, rows_per_w)], idx_all)

        def fire_gathers(s, b):
            for j in range(_KK):
                pltpu.async_copy(
                    w_hbm.at[idx_all.at[s * _KK + j]], row_b.at[b, j],
                    gsem.at[b])

        def drain_gathers(s, b):
            for j in range(_KK):
                pltpu.make_async_copy(
                    w_hbm.at[idx_all.at[s * _KK + j]], row_b.at[b, j],
                    gsem.at[b]).wait()

        def fire_out(s, b):
            pltpu.async_copy(
                row_b.at[b], out_hbm.at[pl.ds(base + s * _KK, _KK)],
                osem.at[b])

        def wait_out(s, b):
            pltpu.make_async_copy(
                row_b.at[b], out_hbm.at[pl.ds(base + s * _KK, _KK)],
                osem.at[b]).wait()

        fire_gathers(0, 0)

        def outer(i, carry):
            s = i * _NB

            @pl.when(i > 0)
            def _free_buf1():
                wait_out(s - 1, 1)

            fire_gathers(s + 1, 1)
            drain_gathers(s, 0)
            fire_out(s, 0)

            wait_out(s, 0)

            @pl.when(i < niter - 1)
            def _refill_buf0():
                fire_gathers(s + 2, 0)

            drain_gathers(s + 1, 1)
            fire_out(s + 1, 1)
            return carry

        lax.fori_loop(0, niter, outer, 0)
        wait_out(steps - 1, 1)

    return emb(tok2, weight)


def kernel(token_ids, weight):
    B, S = token_ids.shape
    V, D = weight.shape
    total = B * S
    R = total // 128
    tok2 = token_ids.reshape(R, 128).astype(jnp.int32)
    out = _embed_sc(tok2, weight, R, D)
    return out.reshape(B, S, D)
